# baseline (speedup 1.0000x reference)
import jax
import jax.numpy as jnp
from jax.experimental import pallas as pl

F_BLK = 256
HW_BLK = 2048


def _matmul_kernel(w_ref, x_ref, o_ref):
    o_ref[0] = jnp.dot(w_ref[...], x_ref[0], preferred_element_type=jnp.float32)


def kernel(inputs, W):
    b, c, h, w = inputs.shape
    filters = W.shape[0]
    hw = h * w
    x = inputs.reshape(b, c, hw)

    grid = (b, hw // HW_BLK, filters // F_BLK)
    out = pl.pallas_call(
        _matmul_kernel,
        grid=grid,
        in_specs=[
            pl.BlockSpec((F_BLK, c), lambda bi, ji, fi: (fi, 0)),
            pl.BlockSpec((1, c, HW_BLK), lambda bi, ji, fi: (bi, 0, ji)),
        ],
        out_specs=pl.BlockSpec((1, F_BLK, HW_BLK), lambda bi, ji, fi: (bi, fi, ji)),
        out_shape=jax.ShapeDtypeStruct((b, filters, hw), jnp.float32),
    )(W, x)
    return out.reshape(b, filters, h, w)


# speedup vs baseline: 1.1441x; 1.1441x over previous
import jax
import jax.numpy as jnp
from jax.experimental import pallas as pl
from jax.experimental.pallas import tpu as pltpu

HW_BLK = 1024


def _matmul_kernel(w_ref, x_ref, o_ref):
    o_ref[0] = jnp.dot(w_ref[...], x_ref[0], preferred_element_type=jnp.float32)


def kernel(inputs, W):
    b, c, h, w = inputs.shape
    filters = W.shape[0]
    hw = h * w
    x = inputs.reshape(b, c, hw)

    grid = (b, hw // HW_BLK)
    out = pl.pallas_call(
        _matmul_kernel,
        grid=grid,
        in_specs=[
            pl.BlockSpec((filters, c), lambda bi, ji: (0, 0)),
            pl.BlockSpec((1, c, HW_BLK), lambda bi, ji: (bi, 0, ji)),
        ],
        out_specs=pl.BlockSpec((1, filters, HW_BLK), lambda bi, ji: (bi, 0, ji)),
        out_shape=jax.ShapeDtypeStruct((b, filters, hw), jnp.float32),
        compiler_params=pltpu.CompilerParams(
            dimension_semantics=("parallel", "parallel"),
        ),
    )(W, x)
    return out.reshape(b, filters, h, w)


# speedup vs baseline: 1.1857x; 1.0364x over previous
import jax
import jax.numpy as jnp
from jax.experimental import pallas as pl
from jax.experimental.pallas import tpu as pltpu

HW_BLK = 2048


def _matmul_kernel(w_ref, x_ref, o_ref):
    o_ref[0] = jnp.dot(w_ref[...], x_ref[0], preferred_element_type=jnp.float32)


def kernel(inputs, W):
    b, c, h, w = inputs.shape
    filters = W.shape[0]
    hw = h * w
    x = inputs.reshape(b, c, hw)

    grid = (b, hw // HW_BLK)
    out = pl.pallas_call(
        _matmul_kernel,
        grid=grid,
        in_specs=[
            pl.BlockSpec((filters, c), lambda bi, ji: (0, 0)),
            pl.BlockSpec((1, c, HW_BLK), lambda bi, ji: (bi, 0, ji)),
        ],
        out_specs=pl.BlockSpec((1, filters, HW_BLK), lambda bi, ji: (bi, 0, ji)),
        out_shape=jax.ShapeDtypeStruct((b, filters, hw), jnp.float32),
        compiler_params=pltpu.CompilerParams(
            dimension_semantics=("parallel", "parallel"),
        ),
    )(W, x)
    return out.reshape(b, filters, h, w)


# speedup vs baseline: 1.1874x; 1.0015x over previous
import jax
import jax.numpy as jnp
from jax.experimental import pallas as pl
from jax.experimental.pallas import tpu as pltpu

HW_BLK = 4096


def _matmul_kernel(w_ref, x_ref, o_ref):
    o_ref[0] = jnp.dot(w_ref[...], x_ref[0], preferred_element_type=jnp.float32)


def kernel(inputs, W):
    b, c, h, w = inputs.shape
    filters = W.shape[0]
    hw = h * w
    x = inputs.reshape(b, c, hw)

    grid = (b, hw // HW_BLK)
    out = pl.pallas_call(
        _matmul_kernel,
        grid=grid,
        in_specs=[
            pl.BlockSpec((filters, c), lambda bi, ji: (0, 0)),
            pl.BlockSpec((1, c, HW_BLK), lambda bi, ji: (bi, 0, ji)),
        ],
        out_specs=pl.BlockSpec((1, filters, HW_BLK), lambda bi, ji: (bi, 0, ji)),
        out_shape=jax.ShapeDtypeStruct((b, filters, hw), jnp.float32),
        compiler_params=pltpu.CompilerParams(
            dimension_semantics=("parallel", "parallel"),
        ),
    )(W, x)
    return out.reshape(b, filters, h, w)
